# baseline (speedup 1.0000x reference)
import functools

import jax
import jax.numpy as jnp
from jax import lax
from jax.experimental import pallas as pl
from jax.experimental.pallas import tpu as pltpu
from jax.experimental.pallas import tpu_sc as plsc

_NC = 2
_NS = 16
_LANES = 16
_CHUNK = 80


@functools.lru_cache(maxsize=None)
def _make_sc_aggregate(n_nodes: int, n_edges: int, d: int):
    nw = _NC * _NS
    assert n_edges % (nw * _CHUNK) == 0
    assert d % _LANES == 0
    e_per_tile = n_edges // nw
    n_chunks = e_per_tile // _CHUNK
    assert n_chunks % 2 == 1
    n_pairs = (n_chunks - 1) // 2
    n_pad = -(-n_nodes // (_NS * 8)) * (_NS * 8)
    rows_per_tile = n_pad // _NS

    mesh = plsc.VectorSubcoreMesh(core_axis_name="c", subcore_axis_name="s",
                                  num_cores=_NC, num_subcores=_NS)

    @functools.partial(
        pl.kernel,
        mesh=mesh,
        compiler_params=pltpu.CompilerParams(use_tc_tiling_on_sc=False),
        out_type=jax.ShapeDtypeStruct((_NC, n_pad, d), jnp.float32),
        scratch_types=[
            pltpu.VMEM_SHARED((n_pad, d), jnp.float32),
            pltpu.VMEM((e_per_tile,), jnp.int32),
            pltpu.VMEM((n_chunks, _CHUNK), jnp.int32),
            pltpu.VMEM((_CHUNK, d), jnp.float32),
            pltpu.VMEM((_CHUNK, d), jnp.float32),
            pltpu.VMEM((_CHUNK,), jnp.float32),
            pltpu.VMEM((_CHUNK,), jnp.float32),
            pltpu.SemaphoreType.DMA,
            pltpu.SemaphoreType.DMA,
            pltpu.SemaphoreType.DMA,
            pltpu.SemaphoreType.DMA,
        ],
    )
    def agg(vals_hbm, col_hbm, row3_hbm, ew_hbm, zeros_hbm, out_hbm,
            acc, col_v, row_2d, rows_a, rows_b, ew_a, ew_b,
            sem_a, sem_b, sem_sa, sem_sb):
        cid = lax.axis_index("c")
        sid = lax.axis_index("s")
        wid = cid * _NS + sid
        ebase = wid * e_per_tile
        rbase = sid * rows_per_tile

        pltpu.sync_copy(zeros_hbm, acc.at[pl.ds(rbase, rows_per_tile)])
        pltpu.sync_copy(col_hbm.at[pl.ds(ebase, e_per_tile)], col_v)
        pltpu.sync_copy(row3_hbm.at[wid], row_2d)
        plsc.subcore_barrier()

        def gather_start(c, buf, ewbuf, sem):
            pltpu.async_copy(
                vals_hbm.at[col_v.at[pl.ds(c * _CHUNK, _CHUNK)]], buf, sem)
            pltpu.async_copy(
                ew_hbm.at[pl.ds(ebase + c * _CHUNK, _CHUNK)], ewbuf, sem)

        def gather_wait(c, buf, ewbuf, sem):
            pltpu.make_async_copy(
                ew_hbm.at[pl.ds(ebase + c * _CHUNK, _CHUNK)], ewbuf, sem
            ).wait()
            pltpu.make_async_copy(
                vals_hbm.at[col_v.at[pl.ds(c * _CHUNK, _CHUNK)]], buf, sem
            ).wait()

        def scale(buf, ewbuf):
            n_groups = _CHUNK // _LANES

            def group_body(g):
                ew16 = ewbuf[pl.ds(g * _LANES, _LANES)]
                for j in range(_LANES):
                    w = ew16.at[jnp.full((_LANES,), j, jnp.int32)].get(
                        mode="promise_in_bounds")
                    e = g * _LANES + j
                    for s in range(d // _LANES):
                        sl = pl.ds(s * _LANES, _LANES)
                        buf[e, sl] = buf[e, sl] * w

            plsc.parallel_loop(0, n_groups, unroll=n_groups)(group_body)

        def scatter_start(c, buf, sem):
            pltpu.async_copy(buf, acc.at[row_2d.at[c]], sem, add=True)

        def scatter_wait(c, buf, sem):
            pltpu.make_async_copy(buf, acc.at[row_2d.at[c]], sem).wait()

        gather_start(0, rows_a, ew_a, sem_a)

        def pair_body(g, carry):
            c0 = 2 * g
            c1 = c0 + 1

            @pl.when(g > 0)
            def _():
                scatter_wait(c0 - 1, rows_b, sem_sb)

            gather_start(c1, rows_b, ew_b, sem_b)
            gather_wait(c0, rows_a, ew_a, sem_a)
            scale(rows_a, ew_a)
            scatter_start(c0, rows_a, sem_sa)
            gather_wait(c1, rows_b, ew_b, sem_b)
            scale(rows_b, ew_b)
            scatter_wait(c0, rows_a, sem_sa)
            gather_start(c1 + 1, rows_a, ew_a, sem_a)
            scatter_start(c1, rows_b, sem_sb)
            return carry

        lax.fori_loop(0, n_pairs, pair_body, 0)
        last = n_chunks - 1
        scatter_wait(last - 1, rows_b, sem_sb)
        gather_wait(last, rows_a, ew_a, sem_a)
        scale(rows_a, ew_a)
        pltpu.sync_copy(rows_a, acc.at[row_2d.at[last]], add=True)

        plsc.subcore_barrier()
        pltpu.sync_copy(
            acc.at[pl.ds(rbase, rows_per_tile)],
            out_hbm.at[cid, pl.ds(rbase, rows_per_tile)],
        )

    def call(vals, col, row, ew, zeros):
        row3 = row.reshape(nw, n_chunks, _CHUNK)
        return agg(vals, col, row3, ew, zeros)

    return call


def _mm_body(parts_ref, w1_ref, b1_ref, w2_ref, out_ref):
    p = parts_ref[0] + parts_ref[1]
    h = jnp.dot(p, w1_ref[...], preferred_element_type=jnp.float32,
                precision=lax.Precision.HIGHEST)
    h = jnp.maximum(h + b1_ref[...], 0.0)
    out_ref[...] = jnp.dot(h, w2_ref[...], preferred_element_type=jnp.float32,
                           precision=lax.Precision.HIGHEST)


def _lsm_body(parts_ref, b2_ref, out_ref):
    a = parts_ref[0] + parts_ref[1] + b2_ref[...]
    m = jnp.max(a, axis=1, keepdims=True)
    ex = jnp.exp(a - m)
    s = jnp.sum(ex, axis=1, keepdims=True)
    out_ref[...] = a - m - jnp.log(s)


def kernel(x, edge_index, edge_weight, W1, b1, W2, b2):
    n, d_in = x.shape
    d_hid = W1.shape[1]
    d_out = W2.shape[1]
    n_edges = edge_weight.shape[0]
    row = edge_index[0]
    col = edge_index[1]
    n_pad = -(-n // (_NS * 8)) * (_NS * 8)

    sc_agg_in = _make_sc_aggregate(n, n_edges, d_in)
    sc_agg_out = _make_sc_aggregate(n, n_edges, d_out)
    zeros_in = jnp.zeros((n_pad // _NS, d_in), jnp.float32)
    zeros_out = jnp.zeros((n_pad // _NS, d_out), jnp.float32)

    parts1 = sc_agg_in(x, col, row, edge_weight, zeros_in)

    bm = n_pad // 8
    assert n_pad % 8 == 0 and bm % 8 == 0
    s2 = pl.pallas_call(
        _mm_body,
        grid=(n_pad // bm,),
        in_specs=[
            pl.BlockSpec((_NC, bm, d_in), lambda i: (0, i, 0)),
            pl.BlockSpec((d_in, d_hid), lambda i: (0, 0)),
            pl.BlockSpec((1, d_hid), lambda i: (0, 0)),
            pl.BlockSpec((d_hid, d_out), lambda i: (0, 0)),
        ],
        out_specs=pl.BlockSpec((bm, d_out), lambda i: (i, 0)),
        out_shape=jax.ShapeDtypeStruct((n_pad, d_out), jnp.float32),
    )(parts1, W1, b1.reshape(1, -1), W2)

    parts2 = sc_agg_out(s2, col, row, edge_weight, zeros_out)

    out = pl.pallas_call(
        _lsm_body,
        grid=(n_pad // bm,),
        in_specs=[
            pl.BlockSpec((_NC, bm, d_out), lambda i: (0, i, 0)),
            pl.BlockSpec((1, d_out), lambda i: (0, 0)),
        ],
        out_specs=pl.BlockSpec((bm, d_out), lambda i: (i, 0)),
        out_shape=jax.ShapeDtypeStruct((n_pad, d_out), jnp.float32),
    )(parts2, b2.reshape(1, -1))
    return out[:n]


# speedup vs baseline: 1.0402x; 1.0402x over previous
import functools

import jax
import jax.numpy as jnp
from jax import lax
from jax.experimental import pallas as pl
from jax.experimental.pallas import tpu as pltpu
from jax.experimental.pallas import tpu_sc as plsc

_NC = 2
_NS = 16
_LANES = 16
_CHUNK = 80


@functools.lru_cache(maxsize=None)
def _make_sc_aggregate(n_nodes: int, n_edges: int, d: int):
    nw = _NC * _NS
    assert n_edges % (nw * _CHUNK) == 0
    assert d % _LANES == 0
    e_per_tile = n_edges // nw
    n_chunks = e_per_tile // _CHUNK
    assert n_chunks % 2 == 1
    n_pairs = (n_chunks - 1) // 2
    n_pad = -(-n_nodes // (_NS * 8)) * (_NS * 8)
    rows_per_tile = n_pad // _NS

    mesh = plsc.VectorSubcoreMesh(core_axis_name="c", subcore_axis_name="s",
                                  num_cores=_NC, num_subcores=_NS)

    @functools.partial(
        pl.kernel,
        mesh=mesh,
        compiler_params=pltpu.CompilerParams(use_tc_tiling_on_sc=False),
        out_type=jax.ShapeDtypeStruct((_NC, n_pad, d), jnp.float32),
        scratch_types=[
            pltpu.VMEM_SHARED((n_pad, d), jnp.float32),
            pltpu.VMEM((e_per_tile,), jnp.int32),
            pltpu.VMEM((n_chunks, _CHUNK), jnp.int32),
            pltpu.VMEM((_CHUNK, d), jnp.float32),
            pltpu.VMEM((_CHUNK, d), jnp.float32),
            pltpu.VMEM((_CHUNK,), jnp.float32),
            pltpu.VMEM((_CHUNK,), jnp.float32),
            pltpu.SemaphoreType.DMA,
            pltpu.SemaphoreType.DMA,
        ],
    )
    def agg(vals_hbm, col_hbm, row3_hbm, ew_hbm, zeros_hbm, out_hbm,
            acc, col_v, row_2d, rows_a, rows_b, ew_a, ew_b, sem_a, sem_b):
        cid = lax.axis_index("c")
        sid = lax.axis_index("s")
        wid = cid * _NS + sid
        ebase = wid * e_per_tile
        rbase = sid * rows_per_tile

        pltpu.sync_copy(zeros_hbm, acc.at[pl.ds(rbase, rows_per_tile)])
        pltpu.sync_copy(col_hbm.at[pl.ds(ebase, e_per_tile)], col_v)
        pltpu.sync_copy(row3_hbm.at[wid], row_2d)
        plsc.subcore_barrier()

        def gather_start(c, buf, ewbuf, sem):
            pltpu.async_copy(
                vals_hbm.at[col_v.at[pl.ds(c * _CHUNK, _CHUNK)]], buf, sem)
            pltpu.async_copy(
                ew_hbm.at[pl.ds(ebase + c * _CHUNK, _CHUNK)], ewbuf, sem)

        def gather_wait(c, buf, ewbuf, sem):
            pltpu.make_async_copy(
                ew_hbm.at[pl.ds(ebase + c * _CHUNK, _CHUNK)], ewbuf, sem
            ).wait()
            pltpu.make_async_copy(
                vals_hbm.at[col_v.at[pl.ds(c * _CHUNK, _CHUNK)]], buf, sem
            ).wait()

        def scale(buf, ewbuf):
            n_groups = _CHUNK // _LANES

            def group_body(g):
                ew16 = ewbuf[pl.ds(g * _LANES, _LANES)]
                for j in range(_LANES):
                    w = ew16.at[jnp.full((_LANES,), j, jnp.int32)].get(
                        mode="promise_in_bounds")
                    e = g * _LANES + j
                    for s in range(d // _LANES):
                        sl = pl.ds(s * _LANES, _LANES)
                        buf[e, sl] = buf[e, sl] * w

            plsc.parallel_loop(0, n_groups, unroll=n_groups)(group_body)

        def scatter(c, buf):
            pltpu.sync_copy(buf, acc.at[row_2d.at[c]], add=True)

        gather_start(0, rows_a, ew_a, sem_a)

        def pair_body(g, carry):
            c0 = 2 * g
            c1 = c0 + 1
            gather_start(c1, rows_b, ew_b, sem_b)
            gather_wait(c0, rows_a, ew_a, sem_a)
            scale(rows_a, ew_a)
            scatter(c0, rows_a)
            gather_start(c1 + 1, rows_a, ew_a, sem_a)
            gather_wait(c1, rows_b, ew_b, sem_b)
            scale(rows_b, ew_b)
            scatter(c1, rows_b)
            return carry

        lax.fori_loop(0, n_pairs, pair_body, 0)
        last = n_chunks - 1
        gather_wait(last, rows_a, ew_a, sem_a)
        scale(rows_a, ew_a)
        scatter(last, rows_a)

        plsc.subcore_barrier()
        pltpu.sync_copy(
            acc.at[pl.ds(rbase, rows_per_tile)],
            out_hbm.at[cid, pl.ds(rbase, rows_per_tile)],
        )

    def call(vals, col, row, ew, zeros):
        row3 = row.reshape(nw, n_chunks, _CHUNK)
        return agg(vals, col, row3, ew, zeros)

    return call


def _mm_body(parts_ref, w1_ref, b1_ref, w2_ref, out_ref):
    p = parts_ref[0] + parts_ref[1]
    h = jnp.dot(p, w1_ref[...], preferred_element_type=jnp.float32,
                precision=lax.Precision.HIGHEST)
    h = jnp.maximum(h + b1_ref[...], 0.0)
    out_ref[...] = jnp.dot(h, w2_ref[...], preferred_element_type=jnp.float32,
                           precision=lax.Precision.HIGHEST)


def _lsm_body(parts_ref, b2_ref, out_ref):
    a = parts_ref[0] + parts_ref[1] + b2_ref[...]
    m = jnp.max(a, axis=1, keepdims=True)
    ex = jnp.exp(a - m)
    s = jnp.sum(ex, axis=1, keepdims=True)
    out_ref[...] = a - m - jnp.log(s)


def kernel(x, edge_index, edge_weight, W1, b1, W2, b2):
    n, d_in = x.shape
    d_hid = W1.shape[1]
    d_out = W2.shape[1]
    n_edges = edge_weight.shape[0]
    row = edge_index[0]
    col = edge_index[1]
    n_pad = -(-n // (_NS * 8)) * (_NS * 8)

    sc_agg_in = _make_sc_aggregate(n, n_edges, d_in)
    sc_agg_out = _make_sc_aggregate(n, n_edges, d_out)
    zeros_in = jnp.zeros((n_pad // _NS, d_in), jnp.float32)
    zeros_out = jnp.zeros((n_pad // _NS, d_out), jnp.float32)

    parts1 = sc_agg_in(x, col, row, edge_weight, zeros_in)

    bm = n_pad // 8
    assert n_pad % 8 == 0 and bm % 8 == 0
    s2 = pl.pallas_call(
        _mm_body,
        grid=(n_pad // bm,),
        in_specs=[
            pl.BlockSpec((_NC, bm, d_in), lambda i: (0, i, 0)),
            pl.BlockSpec((d_in, d_hid), lambda i: (0, 0)),
            pl.BlockSpec((1, d_hid), lambda i: (0, 0)),
            pl.BlockSpec((d_hid, d_out), lambda i: (0, 0)),
        ],
        out_specs=pl.BlockSpec((bm, d_out), lambda i: (i, 0)),
        out_shape=jax.ShapeDtypeStruct((n_pad, d_out), jnp.float32),
    )(parts1, W1, b1.reshape(1, -1), W2)

    parts2 = sc_agg_out(s2, col, row, edge_weight, zeros_out)

    out = pl.pallas_call(
        _lsm_body,
        grid=(n_pad // bm,),
        in_specs=[
            pl.BlockSpec((_NC, bm, d_out), lambda i: (0, i, 0)),
            pl.BlockSpec((1, d_out), lambda i: (0, 0)),
        ],
        out_specs=pl.BlockSpec((bm, d_out), lambda i: (i, 0)),
        out_shape=jax.ShapeDtypeStruct((n_pad, d_out), jnp.float32),
    )(parts2, b2.reshape(1, -1))
    return out[:n]
